# baseline (speedup 1.0000x reference)
import functools

import jax
import jax.numpy as jnp
from jax import lax
from jax.experimental import pallas as pl
from jax.experimental.pallas import tpu as pltpu
from jax.experimental.pallas import tpu_sc as plsc

VOCAB_N = 1000000
TBL = 1048576
CS = 8
NCH = 8
DIM = CS * NCH
BATCH = 16384
ROWS = TBL // CS
NWORK = 32
TPW = BATCH // NWORK
T = 128
SB = TPW // T
WIN = T * NCH
NIV = WIN // 16
NPAIR = 2 * WIN
NIDX = NPAIR // 128
NOV = T * DIM // 16


def _body(x_hbm, t0_hbm, t1_hbm, h0_hbm, h1_hbm, out_hbm,
          x1d, off0, off1, ridx0, ridx1, base0, base1, f0, f1, out1d, sem):
    wid = lax.axis_index("s") * 2 + lax.axis_index("c")
    tok0 = wid * TPW

    lane = lax.iota(jnp.int32, 16)
    lane_hi = lane >> 3
    lane7 = lane & 7
    lane2 = 2 * lane

    pltpu.sync_copy(x_hbm.at[pl.ds(tok0, TPW)], x1d)

    for k in range(SB):
        c0 = pltpu.async_copy(h0_hbm.at[x1d.at[pl.ds(k * T, T)]], off0, sem)
        c1 = pltpu.async_copy(h1_hbm.at[x1d.at[pl.ds(k * T, T)]], off1, sem)
        c0.wait()
        c1.wait()

        def build(i, _):
            row16 = i * 2 + lane_hi
            o0 = plsc.load_gather(off0, [row16, lane7])
            o1 = plsc.load_gather(off1, [row16, lane7])
            r0 = o0 >> 3
            r1 = o1 >> 3
            wbase = (i * 16 + lane) * 16
            base0[pl.ds(i * 16, 16)] = wbase + (o0 & 7)
            base1[pl.ds(i * 16, 16)] = wbase + (o1 & 7)
            q = i * 32 + lane2
            plsc.store_scatter(ridx0, [q], r0)
            plsc.store_scatter(ridx0, [q + 1], (r0 + 1) & (ROWS - 1))
            plsc.store_scatter(ridx1, [q], r1)
            plsc.store_scatter(ridx1, [q + 1], (r1 + 1) & (ROWS - 1))
            return 0

        lax.fori_loop(0, NIV, build, 0)

        copies = []
        for j in range(NIDX):
            copies.append(pltpu.async_copy(
                t0_hbm.at[ridx0.at[pl.ds(j * 128, 128)]],
                f0.at[pl.ds(j * 128, 128)], sem))
            copies.append(pltpu.async_copy(
                t1_hbm.at[ridx1.at[pl.ds(j * 128, 128)]],
                f1.at[pl.ds(j * 128, 128)], sem))
        for c in copies:
            c.wait()

        def assemble(v, _):
            windex = 2 * v + lane_hi
            fi0 = plsc.load_gather(base0, [windex]) + lane7
            fi1 = plsc.load_gather(base1, [windex]) + lane7
            v0 = plsc.load_gather(f0, [fi0 >> 3, fi0 & 7])
            v1 = plsc.load_gather(f1, [fi1 >> 3, fi1 & 7])
            out1d[pl.ds(v * 16, 16)] = v0 + v1
            return 0

        lax.fori_loop(0, NOV, assemble, 0)

        pltpu.sync_copy(out1d, out_hbm.at[pl.ds((tok0 + k * T) * DIM, T * DIM)])


@functools.partial(
    pl.kernel,
    out_type=jax.ShapeDtypeStruct((BATCH * DIM,), jnp.float32),
    mesh=plsc.VectorSubcoreMesh(core_axis_name="c", subcore_axis_name="s",
                                num_cores=2, num_subcores=16),
    compiler_params=pltpu.CompilerParams(
        needs_layout_passes=False, use_tc_tiling_on_sc=False),
    scratch_types=[
        pltpu.VMEM((TPW,), jnp.int32),
        pltpu.VMEM((T, NCH), jnp.int32),
        pltpu.VMEM((T, NCH), jnp.int32),
        pltpu.VMEM((NPAIR,), jnp.int32),
        pltpu.VMEM((NPAIR,), jnp.int32),
        pltpu.VMEM((WIN,), jnp.int32),
        pltpu.VMEM((WIN,), jnp.int32),
        pltpu.VMEM((NPAIR, CS), jnp.float32),
        pltpu.VMEM((NPAIR, CS), jnp.float32),
        pltpu.VMEM((T * DIM,), jnp.float32),
        pltpu.SemaphoreType.DMA,
    ],
)
def _robe_sc(x_hbm, t0_hbm, t1_hbm, h0_hbm, h1_hbm, out_hbm,
             x1d, off0, off1, ridx0, ridx1, base0, base1, f0, f1, out1d, sem):
    _body(x_hbm, t0_hbm, t1_hbm, h0_hbm, h1_hbm, out_hbm,
          x1d, off0, off1, ridx0, ridx1, base0, base1, f0, f1, out1d, sem)


@jax.jit
def kernel(x, table0, table1, h0, h1):
    t0 = table0.reshape(ROWS, CS)
    t1 = table1.reshape(ROWS, CS)
    out = _robe_sc(x.astype(jnp.int32), t0, t1,
                   h0.astype(jnp.int32), h1.astype(jnp.int32))
    return out.reshape(BATCH, DIM)


# speedup vs baseline: 7.3472x; 7.3472x over previous
import functools

import jax
import jax.numpy as jnp
from jax import lax
from jax.experimental import pallas as pl
from jax.experimental.pallas import tpu as pltpu
from jax.experimental.pallas import tpu_sc as plsc

TBL = 1048576
CS = 8
NCH = 8
DIM = CS * NCH
BATCH = 16384
ROWS = TBL // CS
NWORK = 32
TPW = BATCH // NWORK
T = 128
SB = TPW // T
WIN = T * NCH
GWIN = TPW * NCH
NIVALL = GWIN // 16
NPAIR = 2 * WIN
NIDX = NPAIR // 128
NOV = T * DIM // 16


def _body(o0_hbm, o1_hbm, t0_hbm, t1_hbm, out_hbm,
          off0, off1, ridx0, ridx1, base0, base1,
          f0a, f1a, f0b, f1b, outa, outb, semo, semw, semr):
    wid = lax.axis_index("s") * 2 + lax.axis_index("c")
    tok0 = wid * TPW

    lane = lax.iota(jnp.int32, 16)
    lane_hi = lane >> 3
    lane7 = lane & 7
    lane2 = 2 * lane

    doff = [pltpu.async_copy(o0_hbm.at[pl.ds(tok0, TPW)], off0, semo),
            pltpu.async_copy(o1_hbm.at[pl.ds(tok0, TPW)], off1, semo)]
    for d in doff:
        d.wait()

    def build(i, _):
        row16 = i * 2 + lane_hi
        o0 = plsc.load_gather(off0, [row16, lane7])
        o1 = plsc.load_gather(off1, [row16, lane7])
        r0 = o0 >> 3
        r1 = o1 >> 3
        g = i * 16 + lane
        wbase = (g & (WIN - 1)) * 16
        base0[pl.ds(i * 16, 16)] = wbase + (o0 & 7)
        base1[pl.ds(i * 16, 16)] = wbase + (o1 & 7)
        q = i * 32 + lane2
        plsc.store_scatter(ridx0, [q], r0)
        plsc.store_scatter(ridx0, [q + 1], (r0 + 1) & (ROWS - 1))
        plsc.store_scatter(ridx1, [q], r1)
        plsc.store_scatter(ridx1, [q + 1], (r1 + 1) & (ROWS - 1))
        return 0

    lax.fori_loop(0, NIVALL, build, 0, unroll=4)

    def fire_windows(k, fb0, fb1):
        ds = []
        for j in range(NIDX):
            ds.append(pltpu.async_copy(
                t0_hbm.at[ridx0.at[pl.ds(k * NPAIR + j * 128, 128)]],
                fb0.at[pl.ds(j * 128, 128)], semw))
            ds.append(pltpu.async_copy(
                t1_hbm.at[ridx1.at[pl.ds(k * NPAIR + j * 128, 128)]],
                fb1.at[pl.ds(j * 128, 128)], semw))
        return ds

    fbufs = [(f0a, f1a), (f0b, f1b)]
    obufs = [outa, outb]
    pending = fire_windows(0, *fbufs[0])
    dout = [None, None]

    for k in range(SB):
        if k + 1 < SB:
            next_pending = fire_windows(k + 1, *fbufs[(k + 1) & 1])
        for d in pending:
            d.wait()
        if k + 1 < SB:
            pending = next_pending

        fb0, fb1 = fbufs[k & 1]
        ob = obufs[k & 1]
        if dout[k & 1] is not None:
            dout[k & 1].wait()

        kb = k * WIN

        def assemble(v, _):
            windex = kb + 2 * v + lane_hi
            fi0 = plsc.load_gather(base0, [windex]) + lane7
            fi1 = plsc.load_gather(base1, [windex]) + lane7
            v0 = plsc.load_gather(fb0, [fi0 >> 3, fi0 & 7])
            v1 = plsc.load_gather(fb1, [fi1 >> 3, fi1 & 7])
            ob[pl.ds(v * 16, 16)] = v0 + v1
            return 0

        lax.fori_loop(0, NOV, assemble, 0, unroll=8)

        dout[k & 1] = pltpu.async_copy(
            ob, out_hbm.at[pl.ds((tok0 + k * T) * DIM, T * DIM)], semr)

    for d in dout:
        if d is not None:
            d.wait()


@functools.partial(
    pl.kernel,
    out_type=jax.ShapeDtypeStruct((BATCH * DIM,), jnp.float32),
    mesh=plsc.VectorSubcoreMesh(core_axis_name="c", subcore_axis_name="s",
                                num_cores=2, num_subcores=16),
    compiler_params=pltpu.CompilerParams(
        needs_layout_passes=False, use_tc_tiling_on_sc=False),
    scratch_types=[
        pltpu.VMEM((TPW, NCH), jnp.int32),
        pltpu.VMEM((TPW, NCH), jnp.int32),
        pltpu.VMEM((SB * NPAIR,), jnp.int32),
        pltpu.VMEM((SB * NPAIR,), jnp.int32),
        pltpu.VMEM((GWIN,), jnp.int32),
        pltpu.VMEM((GWIN,), jnp.int32),
        pltpu.VMEM((NPAIR, CS), jnp.float32),
        pltpu.VMEM((NPAIR, CS), jnp.float32),
        pltpu.VMEM((NPAIR, CS), jnp.float32),
        pltpu.VMEM((NPAIR, CS), jnp.float32),
        pltpu.VMEM((T * DIM,), jnp.float32),
        pltpu.VMEM((T * DIM,), jnp.float32),
        pltpu.SemaphoreType.DMA,
        pltpu.SemaphoreType.DMA,
        pltpu.SemaphoreType.DMA,
    ],
)
def _robe_sc(o0_hbm, o1_hbm, t0_hbm, t1_hbm, out_hbm,
             off0, off1, ridx0, ridx1, base0, base1,
             f0a, f1a, f0b, f1b, outa, outb, semo, semw, semr):
    _body(o0_hbm, o1_hbm, t0_hbm, t1_hbm, out_hbm,
          off0, off1, ridx0, ridx1, base0, base1,
          f0a, f1a, f0b, f1b, outa, outb, semo, semw, semr)


@jax.jit
def kernel(x, table0, table1, h0, h1):
    t0 = table0.reshape(ROWS, CS)
    t1 = table1.reshape(ROWS, CS)
    off0 = jnp.take(h0, x, axis=0).astype(jnp.int32)
    off1 = jnp.take(h1, x, axis=0).astype(jnp.int32)
    out = _robe_sc(off0, off1, t0, t1)
    return out.reshape(BATCH, DIM)


# speedup vs baseline: 8.3701x; 1.1392x over previous
import functools

import jax
import jax.numpy as jnp
from jax import lax
from jax.experimental import pallas as pl
from jax.experimental.pallas import tpu as pltpu
from jax.experimental.pallas import tpu_sc as plsc

TBL = 1048576
CS = 8
NCH = 8
DIM = CS * NCH
BATCH = 16384
ROWS = TBL // CS
NWORK = 32
TPW = BATCH // NWORK
T = 128
SB = TPW // T
WIN = T * NCH
GWIN = TPW * NCH
NIVALL = GWIN // 16
NPAIR = 2 * WIN
NIDX = NPAIR // 128
NOV = T * DIM // 16


def _dynshuf(a, idx):
    dnums = lax.GatherDimensionNumbers(
        offset_dims=(), collapsed_slice_dims=(0,), start_index_map=(0,))
    return lax.gather(a, idx[:, None], dnums, (1,),
                      mode=lax.GatherScatterMode.PROMISE_IN_BOUNDS)


def _body(o0_hbm, o1_hbm, t0_hbm, t1_hbm, out_hbm,
          off0, off1, ridx0, ridx1, base0, base1,
          f0a, f1a, f0b, f1b, outa, outb, semo, semw, semr):
    wid = lax.axis_index("s") * 2 + lax.axis_index("c")
    tok0 = wid * TPW

    lane = lax.iota(jnp.int32, 16)
    lane_hi = lane >> 3
    lane7 = lane & 7
    lane2 = 2 * lane

    doff = [pltpu.async_copy(o0_hbm.at[pl.ds(tok0 * NCH, GWIN)], off0, semo),
            pltpu.async_copy(o1_hbm.at[pl.ds(tok0 * NCH, GWIN)], off1, semo)]
    for d in doff:
        d.wait()

    def build(i, _):
        o0 = off0[pl.ds(i * 16, 16)]
        o1 = off1[pl.ds(i * 16, 16)]
        r0 = o0 >> 3
        r1 = o1 >> 3
        g = i * 16 + lane
        wbase = (g & (WIN - 1)) * 16
        base0[pl.ds(i * 16, 16)] = wbase + (o0 & 7)
        base1[pl.ds(i * 16, 16)] = wbase + (o1 & 7)
        q = i * 32 + lane2
        plsc.store_scatter(ridx0, [q], r0)
        plsc.store_scatter(ridx0, [q + 1], (r0 + 1) & (ROWS - 1))
        plsc.store_scatter(ridx1, [q], r1)
        plsc.store_scatter(ridx1, [q + 1], (r1 + 1) & (ROWS - 1))
        return 0

    lax.fori_loop(0, NIVALL, build, 0, unroll=4)

    def fire_windows(k, fb0, fb1):
        ds = []
        for j in range(NIDX):
            ds.append(pltpu.async_copy(
                t0_hbm.at[ridx0.at[pl.ds(k * NPAIR + j * 128, 128)]],
                fb0.at[pl.ds(j * 128, 128)], semw))
            ds.append(pltpu.async_copy(
                t1_hbm.at[ridx1.at[pl.ds(k * NPAIR + j * 128, 128)]],
                fb1.at[pl.ds(j * 128, 128)], semw))
        return ds

    fbufs = [(f0a, f1a), (f0b, f1b)]
    obufs = [outa, outb]
    pending = fire_windows(0, *fbufs[0])
    dout = [None, None]

    for k in range(SB):
        if k + 1 < SB:
            next_pending = fire_windows(k + 1, *fbufs[(k + 1) & 1])
        for d in pending:
            d.wait()
        if k + 1 < SB:
            pending = next_pending

        fb0, fb1 = fbufs[k & 1]
        ob = obufs[k & 1]
        if dout[k & 1] is not None:
            dout[k & 1].wait()

        kb = k * GWIN // SB

        def assemble(u, _):
            bv0 = base0[pl.ds(kb + u * 16, 16)]
            bv1 = base1[pl.ds(kb + u * 16, 16)]
            for m in range(8):
                cm = lane_hi + 2 * m
                fi0 = _dynshuf(bv0, cm) + lane7
                fi1 = _dynshuf(bv1, cm) + lane7
                v0 = plsc.load_gather(fb0, [fi0 >> 3, fi0 & 7])
                v1 = plsc.load_gather(fb1, [fi1 >> 3, fi1 & 7])
                ob[pl.ds((u * 8 + m) * 16, 16)] = v0 + v1
            return 0

        lax.fori_loop(0, NOV // 8, assemble, 0, unroll=2)

        dout[k & 1] = pltpu.async_copy(
            ob, out_hbm.at[pl.ds((tok0 + k * T) * DIM, T * DIM)], semr)

    for d in dout:
        if d is not None:
            d.wait()


@functools.partial(
    pl.kernel,
    out_type=jax.ShapeDtypeStruct((BATCH * DIM,), jnp.float32),
    mesh=plsc.VectorSubcoreMesh(core_axis_name="c", subcore_axis_name="s",
                                num_cores=2, num_subcores=16),
    compiler_params=pltpu.CompilerParams(
        needs_layout_passes=False, use_tc_tiling_on_sc=False),
    scratch_types=[
        pltpu.VMEM((GWIN,), jnp.int32),
        pltpu.VMEM((GWIN,), jnp.int32),
        pltpu.VMEM((SB * NPAIR,), jnp.int32),
        pltpu.VMEM((SB * NPAIR,), jnp.int32),
        pltpu.VMEM((GWIN,), jnp.int32),
        pltpu.VMEM((GWIN,), jnp.int32),
        pltpu.VMEM((NPAIR, CS), jnp.float32),
        pltpu.VMEM((NPAIR, CS), jnp.float32),
        pltpu.VMEM((NPAIR, CS), jnp.float32),
        pltpu.VMEM((NPAIR, CS), jnp.float32),
        pltpu.VMEM((T * DIM,), jnp.float32),
        pltpu.VMEM((T * DIM,), jnp.float32),
        pltpu.SemaphoreType.DMA,
        pltpu.SemaphoreType.DMA,
        pltpu.SemaphoreType.DMA,
    ],
)
def _robe_sc(o0_hbm, o1_hbm, t0_hbm, t1_hbm, out_hbm,
             off0, off1, ridx0, ridx1, base0, base1,
             f0a, f1a, f0b, f1b, outa, outb, semo, semw, semr):
    _body(o0_hbm, o1_hbm, t0_hbm, t1_hbm, out_hbm,
          off0, off1, ridx0, ridx1, base0, base1,
          f0a, f1a, f0b, f1b, outa, outb, semo, semw, semr)


@jax.jit
def kernel(x, table0, table1, h0, h1):
    t0 = table0.reshape(ROWS, CS)
    t1 = table1.reshape(ROWS, CS)
    off0 = jnp.take(h0, x, axis=0).astype(jnp.int32).reshape(BATCH * NCH)
    off1 = jnp.take(h1, x, axis=0).astype(jnp.int32).reshape(BATCH * NCH)
    out = _robe_sc(off0, off1, t0, t1)
    return out.reshape(BATCH, DIM)


# speedup vs baseline: 11.6909x; 1.3967x over previous
import functools

import jax
import jax.numpy as jnp
from jax import lax
from jax.experimental import pallas as pl
from jax.experimental.pallas import tpu as pltpu
from jax.experimental.pallas import tpu_sc as plsc

TBL = 1048576
CS = 8
NCH = 8
DIM = CS * NCH
BATCH = 16384
ROWS = TBL // CS
NWORK = 32
TPW = BATCH // NWORK
T = 128
SB = TPW // T
WIN = T * NCH
GWIN = TPW * NCH
NIVALL = GWIN // 16
NPAIR = 2 * WIN
NIDX = NPAIR // 128


def _body(o0_hbm, o1_hbm, t0_hbm, t1_hbm, out_hbm,
          off0, off1, ridx0, ridx1, base0, base1,
          f0a, f1a, f0b, f1b, outa, outb, semo, semw, semr):
    wid = lax.axis_index("s") * 2 + lax.axis_index("c")
    tok0 = wid * TPW

    lane = lax.iota(jnp.int32, 16)
    lane2 = 2 * lane
    lane16 = 16 * lane

    doff = []
    for kk in range(SB):
        doff.append(pltpu.async_copy(
            o0_hbm.at[wid * SB + kk], off0.at[kk], semo))
        doff.append(pltpu.async_copy(
            o1_hbm.at[wid * SB + kk], off1.at[kk], semo))
    for d in doff:
        d.wait()

    def build(i, _):
        kk = i >> 6
        c = (i >> 3) & 7
        tg = i & 7
        o0 = off0[kk, c, pl.ds(tg * 16, 16)]
        o1 = off1[kk, c, pl.ds(tg * 16, 16)]
        r0 = o0 >> 3
        r1 = o1 >> 3
        w16 = (c * 128 + tg * 16) * 16
        bpos = kk * WIN + c * 128 + tg * 16
        base0[pl.ds(bpos, 16)] = w16 + lane16 + (o0 & 7)
        base1[pl.ds(bpos, 16)] = w16 + lane16 + (o1 & 7)
        q = 2 * bpos + lane2
        plsc.store_scatter(ridx0, [q], r0)
        plsc.store_scatter(ridx0, [q + 1], (r0 + 1) & (ROWS - 1))
        plsc.store_scatter(ridx1, [q], r1)
        plsc.store_scatter(ridx1, [q + 1], (r1 + 1) & (ROWS - 1))
        return 0

    def fire_windows(k, fb0, fb1):
        ds = []
        for j in range(NIDX):
            ds.append(pltpu.async_copy(
                t0_hbm.at[ridx0.at[pl.ds(k * NPAIR + j * 128, 128)]],
                fb0.at[pl.ds(j * 128, 128)], semw))
            ds.append(pltpu.async_copy(
                t1_hbm.at[ridx1.at[pl.ds(k * NPAIR + j * 128, 128)]],
                fb1.at[pl.ds(j * 128, 128)], semw))
        return ds

    fbufs = [(f0a, f1a), (f0b, f1b)]
    obufs = [outa, outb]
    lax.fori_loop(0, NIVALL // SB, build, 0, unroll=4)
    pending = fire_windows(0, *fbufs[0])
    lax.fori_loop(NIVALL // SB, NIVALL, build, 0, unroll=4)
    dout = [None, None]

    for k in range(SB):
        if k + 1 < SB:
            next_pending = fire_windows(k + 1, *fbufs[(k + 1) & 1])
        for d in pending:
            d.wait()
        if k + 1 < SB:
            pending = next_pending

        fb0, fb1 = fbufs[k & 1]
        ob = obufs[k & 1]
        if dout[k & 1] is not None:
            dout[k & 1].wait()

        kb = k * WIN

        def assemble(u, _):
            c = u >> 3
            tg = u & 7
            bv0 = base0[pl.ds(kb + c * 128 + tg * 16, 16)]
            bv1 = base1[pl.ds(kb + c * 128 + tg * 16, 16)]
            for j in range(CS):
                fi0 = bv0 + j
                fi1 = bv1 + j
                v0 = plsc.load_gather(fb0, [fi0 >> 3, fi0 & 7])
                v1 = plsc.load_gather(fb1, [fi1 >> 3, fi1 & 7])
                ob[c * 8 + j, pl.ds(tg * 16, 16)] = v0 + v1
            return 0

        lax.fori_loop(0, DIM, assemble, 0, unroll=2)

        dout[k & 1] = pltpu.async_copy(
            ob, out_hbm.at[:, pl.ds(tok0 + k * T, T)], semr)

    for d in dout:
        if d is not None:
            d.wait()


@functools.partial(
    pl.kernel,
    out_type=jax.ShapeDtypeStruct((DIM, BATCH), jnp.float32),
    mesh=plsc.VectorSubcoreMesh(core_axis_name="c", subcore_axis_name="s",
                                num_cores=2, num_subcores=16),
    compiler_params=pltpu.CompilerParams(
        needs_layout_passes=False, use_tc_tiling_on_sc=False),
    scratch_types=[
        pltpu.VMEM((SB, NCH, 128), jnp.int32),
        pltpu.VMEM((SB, NCH, 128), jnp.int32),
        pltpu.VMEM((SB * NPAIR,), jnp.int32),
        pltpu.VMEM((SB * NPAIR,), jnp.int32),
        pltpu.VMEM((GWIN,), jnp.int32),
        pltpu.VMEM((GWIN,), jnp.int32),
        pltpu.VMEM((NPAIR, CS), jnp.float32),
        pltpu.VMEM((NPAIR, CS), jnp.float32),
        pltpu.VMEM((NPAIR, CS), jnp.float32),
        pltpu.VMEM((NPAIR, CS), jnp.float32),
        pltpu.VMEM((DIM, T), jnp.float32),
        pltpu.VMEM((DIM, T), jnp.float32),
        pltpu.SemaphoreType.DMA,
        pltpu.SemaphoreType.DMA,
        pltpu.SemaphoreType.DMA,
    ],
)
def _robe_sc(o0_hbm, o1_hbm, t0_hbm, t1_hbm, out_hbm,
             off0, off1, ridx0, ridx1, base0, base1,
             f0a, f1a, f0b, f1b, outa, outb, semo, semw, semr):
    _body(o0_hbm, o1_hbm, t0_hbm, t1_hbm, out_hbm,
          off0, off1, ridx0, ridx1, base0, base1,
          f0a, f1a, f0b, f1b, outa, outb, semo, semw, semr)


@jax.jit
def kernel(x, table0, table1, h0, h1):
    t0 = table0.reshape(ROWS, CS)
    t1 = table1.reshape(ROWS, CS)
    off0 = (jnp.take(h0, x, axis=0).astype(jnp.int32)
            .T.reshape(NCH, BATCH // 128, 128).transpose(1, 0, 2))
    off1 = (jnp.take(h1, x, axis=0).astype(jnp.int32)
            .T.reshape(NCH, BATCH // 128, 128).transpose(1, 0, 2))
    out = _robe_sc(off0, off1, t0, t1)
    return out.T
